# baseline (speedup 1.0000x reference)
import functools

import jax
import jax.numpy as jnp
from jax import lax
from jax.experimental import pallas as pl
from jax.experimental.pallas import tpu as pltpu
from jax.experimental.pallas import tpu_sc as plsc

NC = 2
NS = 16
LANES = 16

B = 4
T = 2048
D = 128
NW = NC * NS
TPW = T // NW
ROWS_PW = B * TPW


def _body(tok_hbm, x_hbm, pos_hbm, out_hbm, idx_v, tok_v, pos_v, sem):
    wid = lax.axis_index("s") * NC + lax.axis_index("c")
    p0 = wid * TPW

    with jax.named_scope("idx_load"):
        for b in range(B):
            pltpu.sync_copy(x_hbm.at[b, pl.ds(p0, TPW)], idx_v.at[b])

    with jax.named_scope("gather_fire"):
        cps = [
            pltpu.async_copy(
                tok_hbm.at[idx_v.at[b]],
                tok_v.at[pl.ds(b * TPW, TPW)],
                sem,
            )
            for b in range(B)
        ]
    with jax.named_scope("pos_copy"):
        pltpu.sync_copy(pos_hbm.at[pl.ds(p0, TPW)], pos_v)
    with jax.named_scope("gather_drain"):
        for cp in cps:
            cp.wait()

    def add_row(t, carry):
        for j in range(D // LANES):
            sl = pl.ds(j * LANES, LANES)
            p = pos_v[t, sl]
            for b in range(B):
                tok_v[b * TPW + t, sl] = tok_v[b * TPW + t, sl] + p
        return carry

    with jax.named_scope("add_loop"):
        lax.fori_loop(0, TPW, add_row, 0)

    with jax.named_scope("out_store"):
        for b in range(B):
            pltpu.sync_copy(
                tok_v.at[pl.ds(b * TPW, TPW)],
                out_hbm.at[b, pl.ds(p0, TPW)],
            )


@jax.jit
def kernel(x, tok_table, pos_table):
    mesh = plsc.VectorSubcoreMesh(
        core_axis_name="c", subcore_axis_name="s",
        num_cores=NC, num_subcores=NS,
    )
    run = pl.kernel(
        _body,
        out_type=jax.ShapeDtypeStruct((B, T, D), jnp.float32),
        mesh=mesh,
        scratch_types=[
            pltpu.VMEM((B, TPW), jnp.int32),
            pltpu.VMEM((ROWS_PW, D), jnp.float32),
            pltpu.VMEM((TPW, D), jnp.float32),
            pltpu.SemaphoreType.DMA,
        ],
    )
    return run(tok_table, x, pos_table)


# speedup vs baseline: 1.0479x; 1.0479x over previous
import jax
import jax.numpy as jnp
from jax import lax
from jax.experimental import pallas as pl
from jax.experimental.pallas import tpu as pltpu
from jax.experimental.pallas import tpu_sc as plsc

NC = 2
NS = 16
LANES = 16

B = 4
T = 2048
D = 128
NW = NC * NS
TPW = T // NW


def _body(tok_hbm, x_hbm, pos_hbm, out_hbm,
          idx_v, tok_v, pos_v, pos_sem, idx_sem, g_sems, st_sem):
    wid = lax.axis_index("s") * NC + lax.axis_index("c")
    p0 = wid * TPW

    with jax.named_scope("pos_idx"):
        pos_cp = pltpu.async_copy(pos_hbm.at[pl.ds(p0, TPW)], pos_v, pos_sem)
        icps = [
            pltpu.async_copy(x_hbm.at[b, pl.ds(p0, TPW)], idx_v.at[b], idx_sem)
            for b in range(B)
        ]
        for cp in icps:
            cp.wait()

    with jax.named_scope("gather_fire"):
        gcps = [
            pltpu.async_copy(tok_hbm.at[idx_v.at[b]], tok_v.at[b], g_sems[b])
            for b in range(B)
        ]
    with jax.named_scope("pos_wait"):
        pos_cp.wait()

    st_cps = []
    for b in range(B):
        with jax.named_scope("gather_wait"):
            gcps[b].wait()

        def add_row(t, carry, b=b):
            for j in range(D // LANES):
                sl = pl.ds(j * LANES, LANES)
                tok_v[b, t, sl] = tok_v[b, t, sl] + pos_v[t, sl]
            return carry

        with jax.named_scope("add_loop"):
            lax.fori_loop(0, TPW, add_row, 0)
        with jax.named_scope("store_fire"):
            st_cps.append(
                pltpu.async_copy(
                    tok_v.at[b], out_hbm.at[b, pl.ds(p0, TPW)], st_sem
                )
            )

    with jax.named_scope("store_drain"):
        for cp in st_cps:
            cp.wait()


@jax.jit
def kernel(x, tok_table, pos_table):
    mesh = plsc.VectorSubcoreMesh(
        core_axis_name="c", subcore_axis_name="s",
        num_cores=NC, num_subcores=NS,
    )
    run = pl.kernel(
        _body,
        out_type=jax.ShapeDtypeStruct((B, T, D), jnp.float32),
        mesh=mesh,
        scratch_types=[
            pltpu.VMEM((B, TPW), jnp.int32),
            pltpu.VMEM((B, TPW, D), jnp.float32),
            pltpu.VMEM((TPW, D), jnp.float32),
            pltpu.SemaphoreType.DMA,
            pltpu.SemaphoreType.DMA,
            [pltpu.SemaphoreType.DMA] * B,
            pltpu.SemaphoreType.DMA,
        ],
    )
    return run(tok_table, x, pos_table)
